# Initial kernel scaffold; baseline (speedup 1.0000x reference)
import jax
import jax.numpy as jnp
from jax.experimental import pallas as pl


def kernel(x_num, x_cat, weight, category_embeddings, category_offsets):
    raise NotImplementedError("write your pallas kernel here")


# speedup vs baseline: 2.7470x; 2.7470x over previous
import functools

import jax
import jax.numpy as jnp
from jax import lax
from jax.experimental import pallas as pl
from jax.experimental.pallas import tpu as pltpu
from jax.experimental.pallas import tpu_sc as plsc

D_NUM = 13
N_CAT = 26
D_TOK = 64
BATCH = 4096
N_TOK = D_NUM + N_CAT

CB = 32
LANES = 16
CATP = 32


def _make_kernel(nw):
    rw = BATCH // nw
    nch = rw // CB
    nc = 2

    mesh = plsc.VectorSubcoreMesh(core_axis_name="c", subcore_axis_name="s")

    @functools.partial(
        pl.kernel,
        out_type=jax.ShapeDtypeStruct((BATCH * N_TOK, D_TOK), jnp.float32),
        mesh=mesh,
        compiler_params=pltpu.CompilerParams(use_tc_tiling_on_sc=False),
        scratch_types=[
            pltpu.VMEM((CB * CATP,), jnp.int32),
            pltpu.VMEM((CB * CATP,), jnp.int32),
            pltpu.VMEM((CB * CATP,), jnp.int32),
            pltpu.VMEM((CB, LANES), jnp.float32),
            pltpu.VMEM((D_NUM, D_TOK), jnp.float32),
            pltpu.VMEM((CB * N_TOK, D_TOK), jnp.float32),
            pltpu.SemaphoreType.DMA,
        ],
    )
    def tok_kernel(xn_hbm, xc_hbm, w_hbm, emb_hbm, off_hbm, out_hbm,
                   xc_v, off_v, idx_v, xn_v, w_v, obuf, gsem):
        wid = lax.axis_index("s") * nc + lax.axis_index("c")
        b0w = wid * rw

        pltpu.sync_copy(off_hbm, off_v)
        pltpu.sync_copy(w_hbm, w_v)

        for ch in range(nch):
            b0 = b0w + ch * CB

            pltpu.sync_copy(xc_hbm.at[pl.ds(b0 * CATP, CB * CATP)], xc_v)
            pltpu.sync_copy(xn_hbm.at[pl.ds(b0, CB)], xn_v)

            def add_body(j, _):
                s = pl.ds(j * LANES, LANES)
                idx_v[s] = xc_v[s] + off_v[s]
                return 0
            lax.fori_loop(0, CB * CATP // LANES, add_body, 0, unroll=4)

            def fire_body(i, _):
                pltpu.make_async_copy(
                    emb_hbm.at[idx_v.at[pl.ds(i * CATP, N_CAT)]],
                    obuf.at[pl.ds(i * N_TOK + D_NUM, N_CAT)],
                    gsem,
                ).start()
                return 0
            lax.fori_loop(0, CB, fire_body, 0)

            def num_body(i, _):
                row = xn_v[i, :]
                for j in range(D_NUM):
                    xn = row[j]
                    for dk in range(D_TOK // LANES):
                        s = pl.ds(dk * LANES, LANES)
                        obuf[i * N_TOK + j, s] = xn * w_v[j, s]
                return 0
            lax.fori_loop(0, CB, num_body, 0)

            def drain_body(i, _):
                pltpu.make_async_copy(
                    emb_hbm.at[idx_v.at[pl.ds(i * CATP, N_CAT)]],
                    obuf.at[pl.ds(i * N_TOK + D_NUM, N_CAT)],
                    gsem,
                ).wait()
                return 0
            lax.fori_loop(0, CB, drain_body, 0)

            pltpu.sync_copy(obuf, out_hbm.at[pl.ds(b0 * N_TOK, CB * N_TOK)])

    return tok_kernel


def kernel(x_num, x_cat, weight, category_embeddings, category_offsets):
    info = plsc.get_sparse_core_info()
    nw = info.num_cores * info.num_subcores

    xn_pad = jnp.pad(x_num, ((0, 0), (0, LANES - D_NUM)))
    xc_flat = jnp.pad(x_cat.astype(jnp.int32),
                      ((0, 0), (0, CATP - N_CAT))).reshape(-1)
    off_pad = jnp.pad(category_offsets.astype(jnp.int32), (0, CATP - N_CAT))
    off_tile = jnp.tile(off_pad, CB)

    out = _make_kernel(nw)(xn_pad, xc_flat, weight,
                           category_embeddings, off_tile)
    return out.reshape(BATCH, N_TOK, D_TOK)
